# Initial kernel scaffold; baseline (speedup 1.0000x reference)
import jax
import jax.numpy as jnp
from jax.experimental import pallas as pl


def kernel(inputs, weight):
    raise NotImplementedError("write your pallas kernel here")


# speedup vs baseline: 3.0776x; 3.0776x over previous
import functools

import jax
import jax.numpy as jnp
from jax import lax
from jax.experimental import pallas as pl
from jax.experimental.pallas import tpu as pltpu
from jax.experimental.pallas import tpu_sc as plsc

NUM_EMB = 100000
D = 128
TOTAL = 4096 * 50
NC, NS = 2, 16
NW = NC * NS
ROWS_PER_W = TOTAL // NW
CH = 128
CHUNKS = ROWS_PER_W // CH


def _body(table_hbm, idx_hbm, out_hbm, idx_v, rows_v, sem):
    wid = lax.axis_index("s") * NC + lax.axis_index("c")
    base = wid * ROWS_PER_W
    pltpu.sync_copy(idx_hbm.at[wid], idx_v)

    @pl.loop(0, CHUNKS)
    def _chunk(j):
        pltpu.async_copy(table_hbm.at[idx_v.at[j]], rows_v, sem).wait()
        pltpu.sync_copy(rows_v, out_hbm.at[pl.ds(base + j * CH, CH)])


_mesh = plsc.VectorSubcoreMesh(core_axis_name="c", subcore_axis_name="s")

_gather = pl.kernel(
    _body,
    out_type=jax.ShapeDtypeStruct((TOTAL, D), jnp.float32),
    mesh=_mesh,
    scratch_types=[
        pltpu.VMEM((CHUNKS, CH), jnp.int32),
        pltpu.VMEM((CH, D), jnp.float32),
        pltpu.SemaphoreType.DMA,
    ],
)


@jax.jit
def kernel(inputs, weight):
    b, l = inputs.shape
    idx = inputs.astype(jnp.int32).reshape(NW, CHUNKS, CH)
    out = _gather(weight, idx)
    return out.reshape(b, l, D)


# speedup vs baseline: 3.4609x; 1.1246x over previous
import jax
import jax.numpy as jnp
from jax import lax
from jax.experimental import pallas as pl
from jax.experimental.pallas import tpu as pltpu
from jax.experimental.pallas import tpu_sc as plsc

D = 128
TOTAL = 4096 * 50
NC, NS = 2, 16
NW = NC * NS
ROWS_PER_W = TOTAL // NW
CH = 128
CHUNKS = ROWS_PER_W // CH
NBUF = 5
LG = 3


def _body(table_hbm, idx_hbm, out_hbm, idx_v, rows_v, *sems):
    sem_g = sems[:NBUF]
    sem_o = sems[NBUF:]
    wid = lax.axis_index("s") * NC + lax.axis_index("c")
    base = wid * ROWS_PER_W
    pltpu.sync_copy(idx_hbm.at[wid], idx_v)

    def start_gather(j, b):
        pltpu.async_copy(table_hbm.at[idx_v.at[j]], rows_v.at[b], sem_g[b])

    def wait_gather(j, b):
        pltpu.make_async_copy(
            table_hbm.at[idx_v.at[j]], rows_v.at[b], sem_g[b]).wait()

    def start_out(j, b):
        pltpu.async_copy(
            rows_v.at[b], out_hbm.at[pl.ds(base + j * CH, CH)], sem_o[b])

    def wait_out(b):
        pltpu.make_async_copy(
            rows_v.at[b], out_hbm.at[pl.ds(base, CH)], sem_o[b]).wait()

    for b in range(LG):
        start_gather(b, b)

    @pl.loop(0, CHUNKS, step=NBUF)
    def _block(g):
        for u in range(NBUF):
            j = g + u
            bm = (u + LG) % NBUF
            m = j + LG

            @pl.when(m >= NBUF)
            def _():
                wait_out(bm)

            @pl.when(m < CHUNKS)
            def _():
                start_gather(m, bm)

            wait_gather(j, u)
            start_out(j, u)

    for c in range(CHUNKS - (NBUF - LG), CHUNKS):
        wait_out(c % NBUF)


_mesh = plsc.VectorSubcoreMesh(core_axis_name="c", subcore_axis_name="s")

_gather = pl.kernel(
    _body,
    out_type=jax.ShapeDtypeStruct((TOTAL, D), jnp.float32),
    mesh=_mesh,
    scratch_types=[
        pltpu.VMEM((CHUNKS, CH), jnp.int32),
        pltpu.VMEM((NBUF, CH, D), jnp.float32),
    ] + [pltpu.SemaphoreType.DMA] * (2 * NBUF),
)


@jax.jit
def kernel(inputs, weight):
    b, l = inputs.shape
    idx = inputs.astype(jnp.int32).reshape(NW, CHUNKS, CH)
    out = _gather(weight, idx)
    return out.reshape(b, l, D)


# speedup vs baseline: 6.1859x; 1.7874x over previous
import jax
import jax.numpy as jnp
from jax import lax
from jax.experimental import pallas as pl
from jax.experimental.pallas import tpu as pltpu
from jax.experimental.pallas import tpu_sc as plsc

B, L, D = 4096, 50, 128
NC, NS = 2, 16
NW = NC * NS
BPW = B // NW
NBUF = 8
LG = 6


def _body(table_hbm, idx_hbm, out_hbm, idx_v, rows_v, *sems):
    sem_g = sems[:NBUF]
    sem_o = sems[NBUF:]
    wid = lax.axis_index("s") * NC + lax.axis_index("c")
    b0 = wid * BPW
    pltpu.sync_copy(idx_hbm.at[pl.ds(b0, BPW)], idx_v)

    def start_gather(j, bb):
        pltpu.async_copy(table_hbm.at[idx_v.at[j]], rows_v.at[bb], sem_g[bb])

    def wait_gather(j, bb):
        pltpu.make_async_copy(
            table_hbm.at[idx_v.at[j]], rows_v.at[bb], sem_g[bb]).wait()

    def start_out(j, bb):
        pltpu.async_copy(rows_v.at[bb], out_hbm.at[b0 + j], sem_o[bb])

    def wait_out(bb):
        pltpu.make_async_copy(
            rows_v.at[bb], out_hbm.at[b0], sem_o[bb]).wait()

    for bb in range(LG):
        start_gather(bb, bb)

    @pl.loop(0, BPW, step=NBUF)
    def _block(g):
        for u in range(NBUF):
            j = g + u
            bm = (u + LG) % NBUF
            m = j + LG

            @pl.when(m >= NBUF)
            def _():
                wait_out(bm)

            @pl.when(m < BPW)
            def _():
                start_gather(m, bm)

            wait_gather(j, u)
            start_out(j, u)

    for c in range(BPW - (NBUF - LG), BPW):
        wait_out(c % NBUF)


_mesh = plsc.VectorSubcoreMesh(core_axis_name="c", subcore_axis_name="s")

_gather = pl.kernel(
    _body,
    out_type=jax.ShapeDtypeStruct((B, L, D), jnp.float32),
    mesh=_mesh,
    compiler_params=pltpu.CompilerParams(use_tc_tiling_on_sc=True),
    scratch_types=[
        pltpu.VMEM((BPW, L), jnp.int32),
        pltpu.VMEM((NBUF, L, D), jnp.float32),
    ] + [pltpu.SemaphoreType.DMA] * (2 * NBUF),
)


@jax.jit
def kernel(inputs, weight):
    return _gather(weight, inputs.astype(jnp.int32))


# speedup vs baseline: 10.9375x; 1.7681x over previous
import jax
import jax.numpy as jnp
from jax import lax
from jax.experimental import pallas as pl
from jax.experimental.pallas import tpu as pltpu
from jax.experimental.pallas import tpu_sc as plsc

B, L, D = 4096, 50, 128
TOTAL = B * L
NC, NS = 2, 16
NW = NC * NS
ROWS_PER_W = TOTAL // NW
CH = 64
CHUNKS = ROWS_PER_W // CH
NBUF = 10
LG = 8


IDXBUF = 56


def _body(table_hbm, idx_hbm, out_hbm, idx_v, rows_v, *sems):
    sem_g = sems[:NBUF]
    sem_o = sems[NBUF:]
    wid = lax.axis_index("s") * NC + lax.axis_index("c")
    base = wid * ROWS_PER_W
    start8 = pl.multiple_of((wid * 50 // 8) * 8, 8)
    off = wid * 50 - start8
    pltpu.sync_copy(idx_hbm.at[pl.ds(start8, IDXBUF)], idx_v)

    def _idx(j):
        return idx_v.at[off + j // 2, pl.ds((j % 2) * CH, CH)]

    def start_gather(j, bb):
        pltpu.async_copy(table_hbm.at[_idx(j)], rows_v.at[bb], sem_g[bb])

    def wait_gather(j, bb):
        pltpu.make_async_copy(
            table_hbm.at[_idx(j)], rows_v.at[bb], sem_g[bb]).wait()

    def start_out(j, bb):
        row = pl.multiple_of(base + j * CH, CH)
        pltpu.async_copy(rows_v.at[bb], out_hbm.at[pl.ds(row, CH)], sem_o[bb])

    def wait_out(bb):
        row = pl.multiple_of(base, CH)
        pltpu.make_async_copy(
            rows_v.at[bb], out_hbm.at[pl.ds(row, CH)], sem_o[bb]).wait()

    for bb in range(LG):
        start_gather(bb, bb)

    @pl.loop(0, CHUNKS, step=NBUF)
    def _block(g):
        for u in range(NBUF):
            j = g + u
            bm = (u + LG) % NBUF
            m = j + LG

            @pl.when(m >= NBUF)
            def _():
                wait_out(bm)

            @pl.when(m < CHUNKS)
            def _():
                start_gather(m, bm)

            wait_gather(j, u)
            start_out(j, u)

    for c in range(CHUNKS - (NBUF - LG), CHUNKS):
        wait_out(c % NBUF)


_mesh = plsc.VectorSubcoreMesh(core_axis_name="c", subcore_axis_name="s")

_gather = pl.kernel(
    _body,
    out_type=jax.ShapeDtypeStruct((TOTAL, D), jnp.float32),
    mesh=_mesh,
    compiler_params=pltpu.CompilerParams(use_tc_tiling_on_sc=True),
    scratch_types=[
        pltpu.VMEM((IDXBUF, 128), jnp.int32),
        pltpu.VMEM((NBUF, CH, D), jnp.float32),
    ] + [pltpu.SemaphoreType.DMA] * (2 * NBUF),
)


@jax.jit
def kernel(inputs, weight):
    idx = inputs.astype(jnp.int32).T.reshape(NW * 50, 128)
    out = _gather(weight, idx)
    return out.reshape(L, B, D).transpose(1, 0, 2)
